# baseline (speedup 1.0000x reference)
import jax
import jax.numpy as jnp
from functools import partial
from jax.experimental import pallas as pl

RADIUS = 4.0
NUM_SAMPLES = 4
SUB = 2


def _sqdist(a, b):
    return jnp.sum((a[:, None, :] - b[None, :, :]) ** 2, axis=-1)


def _fps_single(xyz, npoint):
    n = xyz.shape[0]

    def body(i, carry):
        idxs, dists, far = carry
        idxs = idxs.at[i].set(far)
        d = jnp.sum((xyz - xyz[far]) ** 2, axis=-1)
        dists = jnp.minimum(dists, d)
        far = jnp.argmax(dists).astype(jnp.int32)
        return idxs, dists, far

    idxs = jnp.zeros((npoint,), jnp.int32)
    dists = jnp.full((n,), 1e10, dtype=xyz.dtype)
    idxs, _, _ = jax.lax.fori_loop(0, npoint, body, (idxs, dists, jnp.int32(0)))
    return idxs


def _fps(xyz, npoint):
    return jax.vmap(lambda p: _fps_single(p, npoint))(xyz)


def _gather_pts(xyz, idx):
    return jax.vmap(lambda p, i: p[i])(xyz, idx)


def _bq_single(radius, nsample, xyz, new_xyz):
    n = xyz.shape[0]
    d2 = _sqdist(new_xyz, xyz)
    mask = d2 < radius * radius
    scores = jnp.where(mask, jnp.arange(n)[None, :], n)
    s = jnp.sort(scores, axis=1)[:, :nsample]
    first = s[:, :1]
    idx = jnp.where(s == n, first, s)
    idx = jnp.where(idx == n, 0, idx)
    return idx


def _ball_query(radius, nsample, xyz, new_xyz):
    return jax.vmap(partial(_bq_single, radius, nsample))(xyz, new_xyz)


def _group_feats(feat, idx):
    return jax.vmap(lambda f, i: f[:, i])(feat, idx)


def _rnn_cell(W, b, radius, nsample, P1, X1, states):
    bsz, S, _ = P1.shape
    cout = W.shape[0]
    if states is None:
        P2 = P1
        S2 = jnp.zeros((bsz, cout, S), P1.dtype)
    else:
        P2, S2 = states
    idx = _ball_query(radius, nsample, P2, P1)
    P2g = _group_feats(jnp.transpose(P2, (0, 2, 1)), idx)
    disp = P2g - jnp.transpose(P1, (0, 2, 1))[..., None]
    S2g = _group_feats(S2, idx)
    if X1 is None:
        feat = jnp.concatenate([disp, S2g], axis=1)
    else:
        X1e = jnp.broadcast_to(X1[..., None], X1.shape + (nsample,))
        feat = jnp.concatenate([disp, X1e, S2g], axis=1)
    out = jnp.einsum('oc,bcsk->bosk', W, feat) + b[None, :, None, None]
    return (P1, jnp.max(out, axis=-1))


def _query_group(radius, nsample, xyz, new_xyz, feats):
    idx = _ball_query(radius, nsample, xyz, new_xyz)
    return _group_feats(feats, idx)


def _fp_single(unknown, known, kf):
    d2 = _sqdist(unknown, known)
    negd2, idx = jax.lax.top_k(-d2, 3)
    dist = jnp.sqrt(jnp.maximum(-negd2, 1e-12))
    recip = 1.0 / (dist + 1e-8)
    w = recip / jnp.sum(recip, axis=1, keepdims=True)
    g = kf[:, idx]
    return jnp.sum(g * w[None, :, :], axis=-1)


def _fp(unknown, known, uf, kf):
    interp = jax.vmap(_fp_single)(unknown, known, kf)
    if uf is not None:
        return jnp.concatenate([interp, uf], axis=1)
    return interp


def _passthrough_kernel(x_ref, o_ref):
    o_ref[...] = x_ref[...]


def kernel(xyzs, e1W, e1b, e2W, e2b, e3W, e3b, d1W, d1b, d2W, d2b, d3W, d3b, mW1, mb1, mW2, mb2):
    r = RADIUS
    ns = NUM_SAMPLES
    rg2 = 2 * r / 4 + 1e-6
    rg3 = 4 * r / 4 + 1e-6
    rc1 = 1 * r + 1e-6
    rc2 = 2 * r + 1e-6
    rc3 = 3 * r + 1e-6
    bsz, l, n, _ = xyzs.shape
    frames = [xyzs[:, t] for t in range(l)]
    s1 = s2 = s3 = None

    def step(frame, s1, s2, s3, c1, c2, c3):
        i1 = _fps(frame, n // SUB)
        x1 = _gather_pts(frame, i1)
        s1 = _rnn_cell(c1[0], c1[1], rc1, 3 * ns, x1, None, s1)
        sx1, sf1 = s1
        i2 = _fps(sx1, n // SUB // SUB)
        x2 = _gather_pts(sx1, i2)
        f2 = jnp.max(_query_group(rg2, ns, sx1, x2, sf1), axis=-1)
        s2 = _rnn_cell(c2[0], c2[1], rc2, 2 * ns, x2, f2, s2)
        sx2, sf2 = s2
        i3 = _fps(sx2, n // SUB // SUB // SUB)
        x3 = _gather_pts(sx2, i3)
        f3 = jnp.max(_query_group(rg3, ns, sx2, x3, sf2), axis=-1)
        s3 = _rnn_cell(c3[0], c3[1], rc3, 1 * ns, x3, f3, s3)
        return s1, s2, s3

    for t in range(l // 2):
        s1, s2, s3 = step(frames[t], s1, s2, s3, (e1W, e1b), (e2W, e2b), (e3W, e3b))
    preds = []
    frame = frames[l // 2 - 1]
    for t in range(l // 2, l):
        s1, s2, s3 = step(frame, s1, s2, s3, (d1W, d1b), (d2W, d2b), (d3W, d3b))
        sx1, sf1 = s1
        sx2, sf2 = s2
        sx3, sf3 = s3
        l3f = _fp(sx2, sx3, sf2, sf3)
        l2f = _fp(sx1, sx2, sf1, l3f)
        l1f = _fp(frame, sx1, None, l2f)
        h = jnp.maximum(jnp.einsum('oc,bcn->bon', mW1, l1f) + mb1[None, :, None], 0.0)
        motion = jnp.einsum('oc,bcn->bon', mW2, h) + mb2[None, :, None]
        motion = jnp.transpose(motion, (0, 2, 1))
        frame = frame + motion
        preds.append(frame)
    out = jnp.stack(preds, axis=1)
    return pl.pallas_call(
        _passthrough_kernel,
        out_shape=jax.ShapeDtypeStruct(out.shape, out.dtype),
    )(out)


# speedup vs baseline: 3.1939x; 3.1939x over previous
import jax
import jax.numpy as jnp
from functools import partial
from jax.experimental import pallas as pl
from jax.experimental.pallas import tpu as pltpu

RADIUS = 4.0
NUM_SAMPLES = 4
SUB = 2

_HI = jax.lax.Precision.HIGHEST


def _dot(a, b, dims):
    return jax.lax.dot_general(a, b, (dims, ((), ())), precision=_HI,
                               preferred_element_type=jnp.float32)


def _pairwise_d2(qt, sc):
    return ((qt[:, 0:1] - sc[0:1, :]) ** 2
            + (qt[:, 1:2] - sc[1:2, :]) ** 2
            + (qt[:, 2:3] - sc[2:3, :]) ** 2)


def _select_onehots(d2, radius2, nsample):
    sq, ss = d2.shape
    iota = jax.lax.broadcasted_iota(jnp.int32, (sq, ss), 1)
    maskf = jnp.where(d2 < radius2, 1.0, 0.0)
    onehots = []
    oh0 = None
    for k in range(nsample):
        scores = jnp.where(maskf > 0.0, iota, ss)
        m = jnp.min(scores, axis=1, keepdims=True)
        validf = jnp.where(m < ss, 1.0, 0.0)
        ohf = jnp.where(iota == m, 1.0, 0.0) * validf
        if k == 0:
            ohf = ohf + (1.0 - validf) * jnp.where(iota == 0, 1.0, 0.0)
            oh0 = ohf
        else:
            ohf = ohf + (1.0 - validf) * oh0
        maskf = maskf * (1.0 - ohf)
        onehots.append(ohf)
    return onehots


def _fps_kernel(npoint, n, x_ref, o_ref, acc_ref):
    x = x_ref[0]
    iota = jax.lax.broadcasted_iota(jnp.int32, (1, n), 1)

    def body(i, carry):
        dists, far = carry
        onehot = iota == far
        acc_ref[pl.ds(i, 1)] = onehot.astype(jnp.float32).reshape(1, 1, n)
        sel = jnp.sum(jnp.where(onehot, x, 0.0), axis=1, keepdims=True)
        d = jnp.sum((x - sel) ** 2, axis=0, keepdims=True)
        dists = jnp.minimum(dists, d)
        m = jnp.max(dists)
        far = jnp.min(jnp.where(dists == m, iota, n))
        return dists, far

    dists0 = jnp.full((1, n), 1e10, jnp.float32)
    jax.lax.fori_loop(0, npoint, body, (dists0, jnp.int32(0)))
    acc = acc_ref[...].reshape(npoint, n)
    o_ref[0] = _dot(x, acc, (([1], [1])))


def _fps(pts_c, npoint):
    b, _, n = pts_c.shape
    return pl.pallas_call(
        partial(_fps_kernel, npoint, n),
        grid=(b,),
        in_specs=[pl.BlockSpec((1, 3, n), lambda i: (i, 0, 0))],
        out_specs=pl.BlockSpec((1, 3, npoint), lambda i: (i, 0, 0)),
        out_shape=jax.ShapeDtypeStruct((b, 3, npoint), jnp.float32),
        scratch_shapes=[pltpu.VMEM((npoint, 1, n), jnp.float32)],
    )(pts_c)


def _rnn_kernel(radius2, nsample, cx, p1t_ref, p2c_ref, s2_ref, x1_ref,
                w_ref, b_ref, o_ref):
    p1t = p1t_ref[0]
    p2c = p2c_ref[0]
    s2 = s2_ref[0]
    w = w_ref[...]
    wd = w[:, 0:3]
    ws = w[:, 3 + cx:]
    g = _dot(wd, p2c, (([1], [0]))) + _dot(ws, s2, (([1], [0])))
    h = _dot(wd, p1t, (([1], [1])))
    d2 = _pairwise_d2(p1t, p2c)
    m = None
    for k, oh in enumerate(_select_onehots(d2, radius2, nsample)):
        gk = _dot(g, oh, (([1], [1])))
        m = gk if k == 0 else jnp.maximum(m, gk)
    out = m - h + b_ref[...]
    if cx:
        out = out + _dot(w[:, 3:3 + cx], x1_ref[0], (([1], [0])))
    o_ref[0] = out


def _rnn_cell(W, b, radius, nsample, p1t, p2c, s2, x1):
    bsz, s, _ = p1t.shape
    o = W.shape[0]
    cx = 0 if x1 is None else x1.shape[1]
    c = s2.shape[1]
    specs = [
        pl.BlockSpec((1, s, 3), lambda i: (i, 0, 0)),
        pl.BlockSpec((1, 3, s), lambda i: (i, 0, 0)),
        pl.BlockSpec((1, c, s), lambda i: (i, 0, 0)),
    ]
    args = [p1t, p2c, s2]
    if x1 is None:
        base = partial(_rnn_kernel, radius * radius, nsample, 0)
        kern = lambda r1, r2, r3, rw, rb, ro: base(r1, r2, r3, None, rw, rb, ro)
    else:
        kern = partial(_rnn_kernel, radius * radius, nsample, cx)
        specs.append(pl.BlockSpec((1, cx, s), lambda i: (i, 0, 0)))
        args.append(x1)
    specs.append(pl.BlockSpec(W.shape, lambda i: (0, 0)))
    specs.append(pl.BlockSpec((o, 1), lambda i: (0, 0)))
    args += [W, b.reshape(o, 1)]
    return pl.pallas_call(
        kern,
        grid=(bsz,),
        in_specs=specs,
        out_specs=pl.BlockSpec((1, o, s), lambda i: (i, 0, 0)),
        out_shape=jax.ShapeDtypeStruct((bsz, o, s), jnp.float32),
    )(*args)


def _bqmax_kernel(radius2, nsample, qt_ref, sc_ref, f_ref, o_ref):
    qt = qt_ref[0]
    sc = sc_ref[0]
    f = f_ref[0]
    d2 = _pairwise_d2(qt, sc)
    m = None
    for k, oh in enumerate(_select_onehots(d2, radius2, nsample)):
        gk = _dot(f, oh, (([1], [1])))
        m = gk if k == 0 else jnp.maximum(m, gk)
    o_ref[0] = m


def _bqmax(radius, nsample, qt, sc, f):
    bsz, sq, _ = qt.shape
    ss = sc.shape[2]
    c = f.shape[1]
    return pl.pallas_call(
        partial(_bqmax_kernel, radius * radius, nsample),
        grid=(bsz,),
        in_specs=[
            pl.BlockSpec((1, sq, 3), lambda i: (i, 0, 0)),
            pl.BlockSpec((1, 3, ss), lambda i: (i, 0, 0)),
            pl.BlockSpec((1, c, ss), lambda i: (i, 0, 0)),
        ],
        out_specs=pl.BlockSpec((1, c, sq), lambda i: (i, 0, 0)),
        out_shape=jax.ShapeDtypeStruct((bsz, c, sq), jnp.float32),
    )(qt, sc, f)


def _fp_kernel(ut_ref, kc_ref, kf_ref, o_ref):
    ut = ut_ref[0]
    kc = kc_ref[0]
    kf = kf_ref[0]
    d2 = _pairwise_d2(ut, kc)
    su, sk = d2.shape
    iota = jax.lax.broadcasted_iota(jnp.int32, (su, sk), 1)
    wacc = jnp.zeros((su, sk), jnp.float32)
    rsum = jnp.zeros((su, 1), jnp.float32)
    for _ in range(3):
        mv = jnp.min(d2, axis=1, keepdims=True)
        sel = jnp.min(jnp.where(d2 == mv, iota, sk), axis=1, keepdims=True)
        oh = iota == sel
        dist = jnp.sqrt(jnp.maximum(mv, 1e-12))
        recip = 1.0 / (dist + 1e-8)
        wacc = wacc + jnp.where(oh, recip, 0.0)
        rsum = rsum + recip
        d2 = jnp.where(oh, 1e30, d2)
    wacc = wacc / rsum
    o_ref[0] = _dot(kf, wacc, (([1], [1])))


def _fp(ut, kc, kf):
    bsz, su, _ = ut.shape
    sk = kc.shape[2]
    c = kf.shape[1]
    return pl.pallas_call(
        _fp_kernel,
        grid=(bsz,),
        in_specs=[
            pl.BlockSpec((1, su, 3), lambda i: (i, 0, 0)),
            pl.BlockSpec((1, 3, sk), lambda i: (i, 0, 0)),
            pl.BlockSpec((1, c, sk), lambda i: (i, 0, 0)),
        ],
        out_specs=pl.BlockSpec((1, c, su), lambda i: (i, 0, 0)),
        out_shape=jax.ShapeDtypeStruct((bsz, c, su), jnp.float32),
    )(ut, kc, kf)


def _mlp_kernel(f_ref, w1_ref, b1_ref, w2_ref, b2_ref, fr_ref, o_ref):
    f = f_ref[0]
    h = jnp.maximum(_dot(w1_ref[...], f, (([1], [0]))) + b1_ref[...], 0.0)
    mo = _dot(w2_ref[...], h, (([1], [0]))) + b2_ref[...]
    o_ref[0] = fr_ref[0] + mo


def _mlp(f, w1, b1, w2, b2, frame_c):
    bsz, c, n = f.shape
    o1 = w1.shape[0]
    return pl.pallas_call(
        _mlp_kernel,
        grid=(bsz,),
        in_specs=[
            pl.BlockSpec((1, c, n), lambda i: (i, 0, 0)),
            pl.BlockSpec(w1.shape, lambda i: (0, 0)),
            pl.BlockSpec((o1, 1), lambda i: (0, 0)),
            pl.BlockSpec(w2.shape, lambda i: (0, 0)),
            pl.BlockSpec((3, 1), lambda i: (0, 0)),
            pl.BlockSpec((1, 3, n), lambda i: (i, 0, 0)),
        ],
        out_specs=pl.BlockSpec((1, 3, n), lambda i: (i, 0, 0)),
        out_shape=jax.ShapeDtypeStruct((bsz, 3, n), jnp.float32),
    )(f, w1, b1.reshape(o1, 1), w2, b2.reshape(3, 1), frame_c)


def _ct(pts_c):
    return jnp.transpose(pts_c, (0, 2, 1))


def kernel(xyzs, e1W, e1b, e2W, e2b, e3W, e3b, d1W, d1b, d2W, d2b, d3W, d3b,
           mW1, mb1, mW2, mb2):
    r = RADIUS
    ns = NUM_SAMPLES
    rg2 = 2 * r / 4 + 1e-6
    rg3 = 4 * r / 4 + 1e-6
    rc1 = 1 * r + 1e-6
    rc2 = 2 * r + 1e-6
    rc3 = 3 * r + 1e-6
    bsz, l, n, _ = xyzs.shape
    s1 = s2 = s3 = None

    def step(frame_c, s1, s2, s3, c1, c2, c3):
        x1c = _fps(frame_c, n // SUB)
        x1t = _ct(x1c)
        if s1 is None:
            p2c, f_prev = x1c, jnp.zeros((bsz, c1[0].shape[0], n // SUB),
                                         jnp.float32)
        else:
            p2c, f_prev = s1
        sf1 = _rnn_cell(c1[0], c1[1], rc1, 3 * ns, x1t, p2c, f_prev, None)
        x2c = _fps(x1c, n // SUB // SUB)
        x2t = _ct(x2c)
        f2 = _bqmax(rg2, ns, x2t, x1c, sf1)
        if s2 is None:
            p2c, f_prev = x2c, jnp.zeros((bsz, c2[0].shape[0], n // SUB // SUB),
                                         jnp.float32)
        else:
            p2c, f_prev = s2
        sf2 = _rnn_cell(c2[0], c2[1], rc2, 2 * ns, x2t, p2c, f_prev, f2)
        x3c = _fps(x2c, n // SUB // SUB // SUB)
        x3t = _ct(x3c)
        f3 = _bqmax(rg3, ns, x3t, x2c, sf2)
        if s3 is None:
            p2c, f_prev = x3c, jnp.zeros(
                (bsz, c3[0].shape[0], n // SUB // SUB // SUB), jnp.float32)
        else:
            p2c, f_prev = s3
        sf3 = _rnn_cell(c3[0], c3[1], rc3, 1 * ns, x3t, p2c, f_prev, f3)
        return (x1c, sf1), (x2c, sf2), (x3c, sf3)

    for t in range(l // 2):
        frame_c = jnp.transpose(xyzs[:, t], (0, 2, 1))
        s1, s2, s3 = step(frame_c, s1, s2, s3, (e1W, e1b), (e2W, e2b),
                          (e3W, e3b))

    preds = []
    frame_c = jnp.transpose(xyzs[:, l // 2 - 1], (0, 2, 1))
    for t in range(l // 2, l):
        s1, s2, s3 = step(frame_c, s1, s2, s3, (d1W, d1b), (d2W, d2b),
                          (d3W, d3b))
        (sx1c, sf1), (sx2c, sf2), (sx3c, sf3) = s1, s2, s3
        l3f = jnp.concatenate([_fp(_ct(sx2c), sx3c, sf3), sf2], axis=1)
        l2f = jnp.concatenate([_fp(_ct(sx1c), sx2c, l3f), sf1], axis=1)
        l1f = _fp(_ct(frame_c), sx1c, l2f)
        frame_c = _mlp(l1f, mW1, mb1, mW2, mb2, frame_c)
        preds.append(jnp.transpose(frame_c, (0, 2, 1)))
    return jnp.stack(preds, axis=1)


# speedup vs baseline: 12.6901x; 3.9732x over previous
import jax
import jax.numpy as jnp
from functools import partial
from jax.experimental import pallas as pl
from jax.experimental.pallas import tpu as pltpu

RADIUS = 4.0
NUM_SAMPLES = 4
SUB = 2

_HI = jax.lax.Precision.HIGHEST


def _dot(a, b, dims):
    return jax.lax.dot_general(a, b, (dims, ((), ())), precision=_HI,
                               preferred_element_type=jnp.float32)


def _pairwise_d2(qt, sc):
    return ((qt[:, 0:1] - sc[0:1, :]) ** 2
            + (qt[:, 1:2] - sc[1:2, :]) ** 2
            + (qt[:, 2:3] - sc[2:3, :]) ** 2)


def _select_onehots(d2, radius2, nsample):
    sq, ss = d2.shape
    iota = jax.lax.broadcasted_iota(jnp.int32, (sq, ss), 1)
    maskf = jnp.where(d2 < radius2, 1.0, 0.0)
    onehots = []
    oh0 = None
    for k in range(nsample):
        scores = jnp.where(maskf > 0.0, iota, ss)
        m = jnp.min(scores, axis=1, keepdims=True)
        validf = jnp.where(m < ss, 1.0, 0.0)
        ohf = jnp.where(iota == m, 1.0, 0.0) * validf
        if k == 0:
            ohf = ohf + (1.0 - validf) * jnp.where(iota == 0, 1.0, 0.0)
            oh0 = ohf
        else:
            ohf = ohf + (1.0 - validf) * oh0
        maskf = maskf * (1.0 - ohf)
        onehots.append(ohf)
    return onehots


def _fps_kernel(npoint, x_ref, o_ref):
    m_, _, n = x_ref.shape
    x0 = x_ref[:, 0, :]
    x1 = x_ref[:, 1, :]
    x2 = x_ref[:, 2, :]
    iota = jax.lax.broadcasted_iota(jnp.int32, (m_, n), 1)

    def body(i, carry):
        dists, far = carry
        onehot = iota == far
        s0 = jnp.sum(jnp.where(onehot, x0, 0.0), axis=1, keepdims=True)
        s1 = jnp.sum(jnp.where(onehot, x1, 0.0), axis=1, keepdims=True)
        s2 = jnp.sum(jnp.where(onehot, x2, 0.0), axis=1, keepdims=True)
        o_ref[pl.ds(i, 1)] = jnp.concatenate([s0, s1, s2], axis=1)[None]
        d = (x0 - s0) ** 2 + (x1 - s1) ** 2 + (x2 - s2) ** 2
        dists = jnp.minimum(dists, d)
        mx = jnp.max(dists, axis=1, keepdims=True)
        far = jnp.min(jnp.where(dists == mx, iota, n), axis=1, keepdims=True)
        return dists, far

    dists0 = jnp.full((m_, n), 1e10, jnp.float32)
    far0 = jnp.zeros((m_, 1), jnp.int32)
    jax.lax.fori_loop(0, npoint, body, (dists0, far0))


def _fps(pts_c, npoint):
    m_, _, n = pts_c.shape
    out = pl.pallas_call(
        partial(_fps_kernel, npoint),
        out_shape=jax.ShapeDtypeStruct((npoint, m_, 3), jnp.float32),
    )(pts_c)
    return jnp.transpose(out, (1, 2, 0)), jnp.transpose(out, (1, 0, 2))


def _rnn_kernel(radius2, nsample, cx, p1t_ref, p2c_ref, s2_ref, x1_ref,
                w_ref, b_ref, o_ref):
    p1t = p1t_ref[0]
    p2c = p2c_ref[0]
    s2 = s2_ref[0]
    w = w_ref[...]
    wd = w[:, 0:3]
    ws = w[:, 3 + cx:]
    g = _dot(wd, p2c, (([1], [0]))) + _dot(ws, s2, (([1], [0])))
    h = _dot(wd, p1t, (([1], [1])))
    d2 = _pairwise_d2(p1t, p2c)
    m = None
    for k, oh in enumerate(_select_onehots(d2, radius2, nsample)):
        gk = _dot(g, oh, (([1], [1])))
        m = gk if k == 0 else jnp.maximum(m, gk)
    out = m - h + b_ref[...]
    if cx:
        out = out + _dot(w[:, 3:3 + cx], x1_ref[0], (([1], [0])))
    o_ref[0] = out


def _rnn_cell(W, b, radius, nsample, p1t, p2c, s2, x1):
    bsz, s, _ = p1t.shape
    o = W.shape[0]
    cx = 0 if x1 is None else x1.shape[1]
    c = s2.shape[1]
    specs = [
        pl.BlockSpec((1, s, 3), lambda i: (i, 0, 0)),
        pl.BlockSpec((1, 3, s), lambda i: (i, 0, 0)),
        pl.BlockSpec((1, c, s), lambda i: (i, 0, 0)),
    ]
    args = [p1t, p2c, s2]
    if x1 is None:
        base = partial(_rnn_kernel, radius * radius, nsample, 0)
        kern = lambda r1, r2, r3, rw, rb, ro: base(r1, r2, r3, None, rw, rb, ro)
    else:
        kern = partial(_rnn_kernel, radius * radius, nsample, cx)
        specs.append(pl.BlockSpec((1, cx, s), lambda i: (i, 0, 0)))
        args.append(x1)
    specs.append(pl.BlockSpec(W.shape, lambda i: (0, 0)))
    specs.append(pl.BlockSpec((o, 1), lambda i: (0, 0)))
    args += [W, b.reshape(o, 1)]
    return pl.pallas_call(
        kern,
        grid=(bsz,),
        in_specs=specs,
        out_specs=pl.BlockSpec((1, o, s), lambda i: (i, 0, 0)),
        out_shape=jax.ShapeDtypeStruct((bsz, o, s), jnp.float32),
    )(*args)


def _bqmax_kernel(radius2, nsample, qt_ref, sc_ref, f_ref, o_ref):
    qt = qt_ref[0]
    sc = sc_ref[0]
    f = f_ref[0]
    d2 = _pairwise_d2(qt, sc)
    m = None
    for k, oh in enumerate(_select_onehots(d2, radius2, nsample)):
        gk = _dot(f, oh, (([1], [1])))
        m = gk if k == 0 else jnp.maximum(m, gk)
    o_ref[0] = m


def _bqmax(radius, nsample, qt, sc, f):
    bsz, sq, _ = qt.shape
    ss = sc.shape[2]
    c = f.shape[1]
    return pl.pallas_call(
        partial(_bqmax_kernel, radius * radius, nsample),
        grid=(bsz,),
        in_specs=[
            pl.BlockSpec((1, sq, 3), lambda i: (i, 0, 0)),
            pl.BlockSpec((1, 3, ss), lambda i: (i, 0, 0)),
            pl.BlockSpec((1, c, ss), lambda i: (i, 0, 0)),
        ],
        out_specs=pl.BlockSpec((1, c, sq), lambda i: (i, 0, 0)),
        out_shape=jax.ShapeDtypeStruct((bsz, c, sq), jnp.float32),
    )(qt, sc, f)


def _fp_kernel(ut_ref, kc_ref, kf_ref, o_ref):
    ut = ut_ref[0]
    kc = kc_ref[0]
    kf = kf_ref[0]
    d2 = _pairwise_d2(ut, kc)
    su, sk = d2.shape
    iota = jax.lax.broadcasted_iota(jnp.int32, (su, sk), 1)
    wacc = jnp.zeros((su, sk), jnp.float32)
    rsum = jnp.zeros((su, 1), jnp.float32)
    for _ in range(3):
        mv = jnp.min(d2, axis=1, keepdims=True)
        sel = jnp.min(jnp.where(d2 == mv, iota, sk), axis=1, keepdims=True)
        oh = iota == sel
        dist = jnp.sqrt(jnp.maximum(mv, 1e-12))
        recip = 1.0 / (dist + 1e-8)
        wacc = wacc + jnp.where(oh, recip, 0.0)
        rsum = rsum + recip
        d2 = jnp.where(oh, 1e30, d2)
    wacc = wacc / rsum
    o_ref[0] = _dot(kf, wacc, (([1], [1])))


def _fp(ut, kc, kf):
    bsz, su, _ = ut.shape
    sk = kc.shape[2]
    c = kf.shape[1]
    return pl.pallas_call(
        _fp_kernel,
        grid=(bsz,),
        in_specs=[
            pl.BlockSpec((1, su, 3), lambda i: (i, 0, 0)),
            pl.BlockSpec((1, 3, sk), lambda i: (i, 0, 0)),
            pl.BlockSpec((1, c, sk), lambda i: (i, 0, 0)),
        ],
        out_specs=pl.BlockSpec((1, c, su), lambda i: (i, 0, 0)),
        out_shape=jax.ShapeDtypeStruct((bsz, c, su), jnp.float32),
    )(ut, kc, kf)


def _mlp_kernel(f_ref, w1_ref, b1_ref, w2_ref, b2_ref, fr_ref, o_ref):
    f = f_ref[0]
    h = jnp.maximum(_dot(w1_ref[...], f, (([1], [0]))) + b1_ref[...], 0.0)
    mo = _dot(w2_ref[...], h, (([1], [0]))) + b2_ref[...]
    o_ref[0] = fr_ref[0] + mo


def _mlp(f, w1, b1, w2, b2, frame_c):
    bsz, c, n = f.shape
    o1 = w1.shape[0]
    return pl.pallas_call(
        _mlp_kernel,
        grid=(bsz,),
        in_specs=[
            pl.BlockSpec((1, c, n), lambda i: (i, 0, 0)),
            pl.BlockSpec(w1.shape, lambda i: (0, 0)),
            pl.BlockSpec((o1, 1), lambda i: (0, 0)),
            pl.BlockSpec(w2.shape, lambda i: (0, 0)),
            pl.BlockSpec((3, 1), lambda i: (0, 0)),
            pl.BlockSpec((1, 3, n), lambda i: (i, 0, 0)),
        ],
        out_specs=pl.BlockSpec((1, 3, n), lambda i: (i, 0, 0)),
        out_shape=jax.ShapeDtypeStruct((bsz, 3, n), jnp.float32),
    )(f, w1, b1.reshape(o1, 1), w2, b2.reshape(3, 1), frame_c)


def _ct(pts_c):
    return jnp.transpose(pts_c, (0, 2, 1))


def kernel(xyzs, e1W, e1b, e2W, e2b, e3W, e3b, d1W, d1b, d2W, d2b, d3W, d3b,
           mW1, mb1, mW2, mb2):
    r = RADIUS
    ns = NUM_SAMPLES
    rg2 = 2 * r / 4 + 1e-6
    rg3 = 4 * r / 4 + 1e-6
    rc1 = 1 * r + 1e-6
    rc2 = 2 * r + 1e-6
    rc3 = 3 * r + 1e-6
    bsz, l, n, _ = xyzs.shape
    s1 = s2 = s3 = None

    def step(samples, s1, s2, s3, c1, c2, c3):
        x1c, x1t, x2c, x2t, x3c, x3t = samples
        if s1 is None:
            p2c, f_prev = x1c, jnp.zeros((bsz, c1[0].shape[0], n // SUB),
                                         jnp.float32)
        else:
            p2c, f_prev = s1
        sf1 = _rnn_cell(c1[0], c1[1], rc1, 3 * ns, x1t, p2c, f_prev, None)
        f2 = _bqmax(rg2, ns, x2t, x1c, sf1)
        if s2 is None:
            p2c, f_prev = x2c, jnp.zeros((bsz, c2[0].shape[0], n // SUB // SUB),
                                         jnp.float32)
        else:
            p2c, f_prev = s2
        sf2 = _rnn_cell(c2[0], c2[1], rc2, 2 * ns, x2t, p2c, f_prev, f2)
        f3 = _bqmax(rg3, ns, x3t, x2c, sf2)
        if s3 is None:
            p2c, f_prev = x3c, jnp.zeros(
                (bsz, c3[0].shape[0], n // SUB // SUB // SUB), jnp.float32)
        else:
            p2c, f_prev = s3
        sf3 = _rnn_cell(c3[0], c3[1], rc3, 1 * ns, x3t, p2c, f_prev, f3)
        return (x1c, sf1), (x2c, sf2), (x3c, sf3)

    l2 = l // 2
    enc_c = jnp.transpose(xyzs[:, :l2], (0, 1, 3, 2)).reshape(bsz * l2, 3, n)
    x1c_a, x1t_a = _fps(enc_c, n // SUB)
    x2c_a, x2t_a = _fps(x1c_a, n // SUB // SUB)
    x3c_a, x3t_a = _fps(x2c_a, n // SUB // SUB // SUB)

    def sl(a, t):
        return a.reshape((bsz, l2) + a.shape[1:])[:, t]

    for t in range(l2):
        samples = (sl(x1c_a, t), sl(x1t_a, t), sl(x2c_a, t), sl(x2t_a, t),
                   sl(x3c_a, t), sl(x3t_a, t))
        s1, s2, s3 = step(samples, s1, s2, s3, (e1W, e1b), (e2W, e2b),
                          (e3W, e3b))

    preds = []
    frame_c = jnp.transpose(xyzs[:, l2 - 1], (0, 2, 1))
    for t in range(l2, l):
        x1c, x1t = _fps(frame_c, n // SUB)
        x2c, x2t = _fps(x1c, n // SUB // SUB)
        x3c, x3t = _fps(x2c, n // SUB // SUB // SUB)
        samples = (x1c, x1t, x2c, x2t, x3c, x3t)
        s1, s2, s3 = step(samples, s1, s2, s3, (d1W, d1b), (d2W, d2b),
                          (d3W, d3b))
        (sx1c, sf1), (sx2c, sf2), (sx3c, sf3) = s1, s2, s3
        l3f = jnp.concatenate([_fp(_ct(sx2c), sx3c, sf3), sf2], axis=1)
        l2f = jnp.concatenate([_fp(_ct(sx1c), sx2c, l3f), sf1], axis=1)
        l1f = _fp(_ct(frame_c), sx1c, l2f)
        frame_c = _mlp(l1f, mW1, mb1, mW2, mb2, frame_c)
        preds.append(jnp.transpose(frame_c, (0, 2, 1)))
    return jnp.stack(preds, axis=1)


# speedup vs baseline: 13.3762x; 1.0541x over previous
import jax
import jax.numpy as jnp
from functools import partial
from jax.experimental import pallas as pl

RADIUS = 4.0
NUM_SAMPLES = 4
SUB = 2

_HI = jax.lax.Precision.HIGHEST


def _dot(a, b, dims):
    return jax.lax.dot_general(a, b, (dims, ((), ())), precision=_HI,
                               preferred_element_type=jnp.float32)


def _pairwise_d2(qt, sc):
    return ((qt[:, 0:1] - sc[0:1, :]) ** 2
            + (qt[:, 1:2] - sc[1:2, :]) ** 2
            + (qt[:, 2:3] - sc[2:3, :]) ** 2)


def _select_onehots(d2, radius2, nsample):
    sq, ss = d2.shape
    iota = jax.lax.broadcasted_iota(jnp.int32, (sq, ss), 1)
    maskf = jnp.where(d2 < radius2, 1.0, 0.0)
    tri_i = jax.lax.broadcasted_iota(jnp.int32, (ss, ss), 0)
    tri_j = jax.lax.broadcasted_iota(jnp.int32, (ss, ss), 1)
    tri = jnp.where(tri_i <= tri_j, 1.0, 0.0)
    rank_incl = _dot(maskf, tri, (([1], [0])))
    rank = jnp.where(maskf > 0.0, rank_incl - maskf, -1.0)
    count = rank_incl[:, ss - 1:ss]
    e0 = jnp.where(iota == 0, 1.0, 0.0)
    oh0 = jnp.where(rank == 0.0, 1.0, 0.0) \
        + jnp.where(count > 0.0, 0.0, 1.0) * e0
    onehots = [oh0]
    for k in range(1, nsample):
        ohk = jnp.where(rank == float(k), 1.0, 0.0)
        validf = jnp.where(count > float(k), 1.0, 0.0)
        onehots.append(ohk + (1.0 - validf) * oh0)
    return onehots


def _maxgather(feats, d2, radius2, nsample):
    m = None
    for oh in _select_onehots(d2, radius2, nsample):
        gk = _dot(feats, oh, (([1], [1])))
        m = gk if m is None else jnp.maximum(m, gk)
    return m


def _rnn_core(radius2, nsample, p1t, p2c, s2, x1, w, b):
    cx = 0 if x1 is None else x1.shape[0]
    wd = w[:, 0:3]
    ws = w[:, 3 + cx:]
    g = _dot(wd, p2c, (([1], [0]))) + _dot(ws, s2, (([1], [0])))
    h = _dot(wd, p1t, (([1], [1])))
    d2 = _pairwise_d2(p1t, p2c)
    out = _maxgather(g, d2, radius2, nsample) - h + b
    if x1 is not None:
        out = out + _dot(w[:, 3:3 + cx], x1, (([1], [0])))
    return out


def _fp_core(ut, kc, kf):
    d2 = _pairwise_d2(ut, kc)
    su, sk = d2.shape
    iota = jax.lax.broadcasted_iota(jnp.int32, (su, sk), 1)
    wacc = jnp.zeros((su, sk), jnp.float32)
    rsum = jnp.zeros((su, 1), jnp.float32)
    for _ in range(3):
        mv = jnp.min(d2, axis=1, keepdims=True)
        sel = jnp.min(jnp.where(d2 == mv, iota, sk), axis=1, keepdims=True)
        oh = iota == sel
        dist = jnp.sqrt(jnp.maximum(mv, 1e-12))
        recip = 1.0 / (dist + 1e-8)
        wacc = wacc + jnp.where(oh, recip, 0.0)
        rsum = rsum + recip
        d2 = jnp.where(oh, 1e30, d2)
    wacc = wacc / rsum
    return _dot(kf, wacc, (([1], [1])))


def _fps_kernel(npoint, x_ref, o_ref):
    m_, _, n = x_ref.shape
    x0 = x_ref[:, 0, :]
    x1 = x_ref[:, 1, :]
    x2 = x_ref[:, 2, :]
    iota = jax.lax.broadcasted_iota(jnp.int32, (m_, n), 1)

    def body(i, carry):
        dists, far = carry
        onehot = iota == far
        s0 = jnp.sum(jnp.where(onehot, x0, 0.0), axis=1, keepdims=True)
        s1 = jnp.sum(jnp.where(onehot, x1, 0.0), axis=1, keepdims=True)
        s2 = jnp.sum(jnp.where(onehot, x2, 0.0), axis=1, keepdims=True)
        o_ref[pl.ds(i, 1)] = jnp.concatenate([s0, s1, s2], axis=1)[None]
        d = (x0 - s0) ** 2 + (x1 - s1) ** 2 + (x2 - s2) ** 2
        dists = jnp.minimum(dists, d)
        mx = jnp.max(dists, axis=1, keepdims=True)
        far = jnp.min(jnp.where(dists == mx, iota, n), axis=1, keepdims=True)
        return dists, far

    dists0 = jnp.full((m_, n), 1e10, jnp.float32)
    far0 = jnp.zeros((m_, 1), jnp.int32)
    jax.lax.fori_loop(0, npoint, body, (dists0, far0))


def _fps(pts_c, npoint):
    m_, _, n = pts_c.shape
    out = pl.pallas_call(
        partial(_fps_kernel, npoint),
        out_shape=jax.ShapeDtypeStruct((npoint, m_, 3), jnp.float32),
    )(pts_c)
    return jnp.transpose(out, (1, 2, 0)), jnp.transpose(out, (1, 0, 2))


def _cells_kernel(cfg, x1t_ref, x1c_ref, x2t_ref, x2c_ref, x3t_ref, x3c_ref,
                  p1_ref, f1_ref, p2_ref, f2_ref, p3_ref, f3_ref,
                  w1_ref, b1_ref, w2_ref, b2_ref, w3_ref, b3_ref,
                  o1_ref, o2_ref, o3_ref):
    rc1sq, rc2sq, rc3sq, rg2sq, rg3sq, ns = cfg
    sf1 = _rnn_core(rc1sq, 3 * ns, x1t_ref[0], p1_ref[0], f1_ref[0], None,
                    w1_ref[...], b1_ref[...])
    o1_ref[0] = sf1
    d2 = _pairwise_d2(x2t_ref[0], x1c_ref[0])
    f2 = _maxgather(sf1, d2, rg2sq, ns)
    sf2 = _rnn_core(rc2sq, 2 * ns, x2t_ref[0], p2_ref[0], f2_ref[0], f2,
                    w2_ref[...], b2_ref[...])
    o2_ref[0] = sf2
    d2 = _pairwise_d2(x3t_ref[0], x2c_ref[0])
    f3 = _maxgather(sf2, d2, rg3sq, ns)
    sf3 = _rnn_core(rc3sq, 1 * ns, x3t_ref[0], p3_ref[0], f3_ref[0], f3,
                    w3_ref[...], b3_ref[...])
    o3_ref[0] = sf3


def _cells(cfg, samples, states, weights):
    x1c, x1t, x2c, x2t, x3c, x3t = samples
    (p1, f1), (p2, f2), (p3, f3) = states
    w1, b1, w2, b2, w3, b3 = weights
    bsz = x1t.shape[0]
    arrs = [x1t, x1c, x2t, x2c, x3t, x3c, p1, f1, p2, f2, p3, f3]
    specs = [pl.BlockSpec((1,) + a.shape[1:],
                          lambda i, nd=a.ndim: (i,) + (0,) * (nd - 1))
             for a in arrs]
    for w, b in ((w1, b1), (w2, b2), (w3, b3)):
        arrs += [w, b.reshape(-1, 1)]
        specs += [pl.BlockSpec(w.shape, lambda i: (0, 0)),
                  pl.BlockSpec((b.shape[0], 1), lambda i: (0, 0))]
    outs = [jax.ShapeDtypeStruct((bsz, w.shape[0], p.shape[2]), jnp.float32)
            for w, p in ((w1, p1), (w2, p2), (w3, p3))]
    out_specs = [pl.BlockSpec((1,) + o.shape[1:], lambda i: (i, 0, 0))
                 for o in outs]
    return pl.pallas_call(
        partial(_cells_kernel, cfg),
        grid=(bsz,),
        in_specs=specs,
        out_specs=out_specs,
        out_shape=outs,
    )(*arrs)


def _fpmlp_kernel(x2t_ref, x3c_ref, sf3_ref, sf2_ref, x1t_ref, x2c_ref,
                  sf1_ref, ft_ref, x1c_ref, fc_ref,
                  w1_ref, b1_ref, w2_ref, b2_ref, o_ref):
    l3f = jnp.concatenate(
        [_fp_core(x2t_ref[0], x3c_ref[0], sf3_ref[0]), sf2_ref[0]], axis=0)
    l2f = jnp.concatenate(
        [_fp_core(x1t_ref[0], x2c_ref[0], l3f), sf1_ref[0]], axis=0)
    l1f = _fp_core(ft_ref[0], x1c_ref[0], l2f)
    h = jnp.maximum(_dot(w1_ref[...], l1f, (([1], [0]))) + b1_ref[...], 0.0)
    mo = _dot(w2_ref[...], h, (([1], [0]))) + b2_ref[...]
    o_ref[0] = fc_ref[0] + mo


def _fpmlp(samples, states, frame_t, frame_c, mw1, mb1, mw2, mb2):
    x1c, x1t, x2c, x2t, _, _ = samples
    (_, sf1), (sx2c, sf2), (sx3c, sf3) = states
    bsz, _, n = frame_c.shape
    arrs = [x2t, sx3c, sf3, sf2, x1t, sx2c, sf1, frame_t, x1c, frame_c]
    specs = [pl.BlockSpec((1,) + a.shape[1:],
                          lambda i, nd=a.ndim: (i,) + (0,) * (nd - 1))
             for a in arrs]
    arrs += [mw1, mb1.reshape(-1, 1), mw2, mb2.reshape(-1, 1)]
    specs += [pl.BlockSpec(mw1.shape, lambda i: (0, 0)),
              pl.BlockSpec((mb1.shape[0], 1), lambda i: (0, 0)),
              pl.BlockSpec(mw2.shape, lambda i: (0, 0)),
              pl.BlockSpec((mb2.shape[0], 1), lambda i: (0, 0))]
    return pl.pallas_call(
        _fpmlp_kernel,
        grid=(bsz,),
        in_specs=specs,
        out_specs=pl.BlockSpec((1, 3, n), lambda i: (i, 0, 0)),
        out_shape=jax.ShapeDtypeStruct((bsz, 3, n), jnp.float32),
    )(*arrs)


def _ct(pts_c):
    return jnp.transpose(pts_c, (0, 2, 1))


def kernel(xyzs, e1W, e1b, e2W, e2b, e3W, e3b, d1W, d1b, d2W, d2b, d3W, d3b,
           mW1, mb1, mW2, mb2):
    r = RADIUS
    ns = NUM_SAMPLES
    rg2 = 2 * r / 4 + 1e-6
    rg3 = 4 * r / 4 + 1e-6
    rc1 = 1 * r + 1e-6
    rc2 = 2 * r + 1e-6
    rc3 = 3 * r + 1e-6
    cfg = (rc1 * rc1, rc2 * rc2, rc3 * rc3, rg2 * rg2, rg3 * rg3, ns)
    bsz, l, n, _ = xyzs.shape
    n1, n2, n3 = n // SUB, n // SUB // SUB, n // SUB // SUB // SUB

    def zero_states(samples):
        x1c, _, x2c, _, x3c, _ = samples
        return ((x1c, jnp.zeros((bsz, e1W.shape[0], n1), jnp.float32)),
                (x2c, jnp.zeros((bsz, e2W.shape[0], n2), jnp.float32)),
                (x3c, jnp.zeros((bsz, e3W.shape[0], n3), jnp.float32)))

    def step(samples, states, weights):
        sf1, sf2, sf3 = _cells(cfg, samples, states, weights)
        x1c, _, x2c, _, x3c, _ = samples
        return ((x1c, sf1), (x2c, sf2), (x3c, sf3))

    l2 = l // 2
    enc_c = jnp.transpose(xyzs[:, :l2], (0, 1, 3, 2)).reshape(bsz * l2, 3, n)
    x1c_a, x1t_a = _fps(enc_c, n1)
    x2c_a, x2t_a = _fps(x1c_a, n2)
    x3c_a, x3t_a = _fps(x2c_a, n3)

    def sl(a, t):
        return a.reshape((bsz, l2) + a.shape[1:])[:, t]

    enc_w = (e1W, e1b, e2W, e2b, e3W, e3b)
    dec_w = (d1W, d1b, d2W, d2b, d3W, d3b)
    states = None
    for t in range(l2):
        samples = (sl(x1c_a, t), sl(x1t_a, t), sl(x2c_a, t), sl(x2t_a, t),
                   sl(x3c_a, t), sl(x3t_a, t))
        if states is None:
            states = zero_states(samples)
        states = step(samples, states, enc_w)

    preds = []
    frame_c = jnp.transpose(xyzs[:, l2 - 1], (0, 2, 1))
    for t in range(l2, l):
        x1c, x1t = _fps(frame_c, n1)
        x2c, x2t = _fps(x1c, n2)
        x3c, x3t = _fps(x2c, n3)
        samples = (x1c, x1t, x2c, x2t, x3c, x3t)
        states = step(samples, states, dec_w)
        frame_c = _fpmlp(samples, states, _ct(frame_c), frame_c,
                         mW1, mb1, mW2, mb2)
        preds.append(_ct(frame_c))
    return jnp.stack(preds, axis=1)


# speedup vs baseline: 15.8695x; 1.1864x over previous
import jax
import jax.numpy as jnp
from functools import partial
from jax.experimental import pallas as pl

RADIUS = 4.0
NUM_SAMPLES = 4
SUB = 2

_HI = jax.lax.Precision.HIGHEST


def _dot(a, b, dims):
    return jax.lax.dot_general(a, b, (dims, ((), ())), precision=_HI,
                               preferred_element_type=jnp.float32)


def _pairwise_d2(qt, sc):
    return ((qt[:, 0:1] - sc[0:1, :]) ** 2
            + (qt[:, 1:2] - sc[1:2, :]) ** 2
            + (qt[:, 2:3] - sc[2:3, :]) ** 2)


def _select_onehots(d2, radius2, nsample):
    sq, ss = d2.shape
    iota = jax.lax.broadcasted_iota(jnp.int32, (sq, ss), 1)
    maskf = jnp.where(d2 < radius2, 1.0, 0.0)
    tri_i = jax.lax.broadcasted_iota(jnp.int32, (ss, ss), 0)
    tri_j = jax.lax.broadcasted_iota(jnp.int32, (ss, ss), 1)
    tri = jnp.where(tri_i <= tri_j, 1.0, 0.0)
    rank_incl = _dot(maskf, tri, (([1], [0])))
    rank = jnp.where(maskf > 0.0, rank_incl - maskf, -1.0)
    count = rank_incl[:, ss - 1:ss]
    e0 = jnp.where(iota == 0, 1.0, 0.0)
    oh0 = jnp.where(rank == 0.0, 1.0, 0.0) \
        + jnp.where(count > 0.0, 0.0, 1.0) * e0
    onehots = [oh0]
    for k in range(1, nsample):
        ohk = jnp.where(rank == float(k), 1.0, 0.0)
        validf = jnp.where(count > float(k), 1.0, 0.0)
        onehots.append(ohk + (1.0 - validf) * oh0)
    return onehots


def _maxgather(feats, d2, radius2, nsample):
    m = None
    for oh in _select_onehots(d2, radius2, nsample):
        gk = _dot(feats, oh, (([1], [1])))
        m = gk if m is None else jnp.maximum(m, gk)
    return m


def _rnn_core(radius2, nsample, p1t, p2c, s2, x1, w, b):
    cx = 0 if x1 is None else x1.shape[0]
    wd = w[:, 0:3]
    ws = w[:, 3 + cx:]
    g = _dot(wd, p2c, (([1], [0]))) + _dot(ws, s2, (([1], [0])))
    h = _dot(wd, p1t, (([1], [1])))
    d2 = _pairwise_d2(p1t, p2c)
    out = _maxgather(g, d2, radius2, nsample) - h + b
    if x1 is not None:
        out = out + _dot(w[:, 3:3 + cx], x1, (([1], [0])))
    return out


def _fp_core(ut, kc, kf):
    d2 = _pairwise_d2(ut, kc)
    su, sk = d2.shape
    iota = jax.lax.broadcasted_iota(jnp.int32, (su, sk), 1)
    wacc = jnp.zeros((su, sk), jnp.float32)
    rsum = jnp.zeros((su, 1), jnp.float32)
    for _ in range(3):
        mv = jnp.min(d2, axis=1, keepdims=True)
        sel = jnp.min(jnp.where(d2 == mv, iota, sk), axis=1, keepdims=True)
        oh = iota == sel
        dist = jnp.sqrt(jnp.maximum(mv, 1e-12))
        recip = 1.0 / (dist + 1e-8)
        wacc = wacc + jnp.where(oh, recip, 0.0)
        rsum = rsum + recip
        d2 = jnp.where(oh, 1e30, d2)
    wacc = wacc / rsum
    return _dot(kf, wacc, (([1], [1])))


def _fps_kernel(npoint, x_ref, o_ref):
    m_, _, n = x_ref.shape
    x0 = x_ref[:, 0, :]
    x1 = x_ref[:, 1, :]
    x2 = x_ref[:, 2, :]
    iota = jax.lax.broadcasted_iota(jnp.int32, (m_, n), 1)

    def body(i, carry):
        dists, far = carry
        onehot = iota == far
        s0 = jnp.sum(jnp.where(onehot, x0, 0.0), axis=1, keepdims=True)
        s1 = jnp.sum(jnp.where(onehot, x1, 0.0), axis=1, keepdims=True)
        s2 = jnp.sum(jnp.where(onehot, x2, 0.0), axis=1, keepdims=True)
        o_ref[pl.ds(i, 1)] = jnp.concatenate([s0, s1, s2], axis=1)[None]
        d = (x0 - s0) ** 2 + (x1 - s1) ** 2 + (x2 - s2) ** 2
        dists = jnp.minimum(dists, d)
        far = jnp.argmax(dists, axis=1, keepdims=True).astype(jnp.int32)
        return dists, far

    dists0 = jnp.full((m_, n), 1e10, jnp.float32)
    far0 = jnp.zeros((m_, 1), jnp.int32)
    jax.lax.fori_loop(0, npoint, body, (dists0, far0))


def _fps(pts_c, npoint):
    m_, _, n = pts_c.shape
    out = pl.pallas_call(
        partial(_fps_kernel, npoint),
        out_shape=jax.ShapeDtypeStruct((npoint, m_, 3), jnp.float32),
    )(pts_c)
    return jnp.transpose(out, (1, 2, 0)), jnp.transpose(out, (1, 0, 2))


def _cells_kernel(cfg, x1t_ref, x1c_ref, x2t_ref, x2c_ref, x3t_ref, x3c_ref,
                  p1_ref, f1_ref, p2_ref, f2_ref, p3_ref, f3_ref,
                  w1_ref, b1_ref, w2_ref, b2_ref, w3_ref, b3_ref,
                  o1_ref, o2_ref, o3_ref):
    rc1sq, rc2sq, rc3sq, rg2sq, rg3sq, ns = cfg
    sf1 = _rnn_core(rc1sq, 3 * ns, x1t_ref[0], p1_ref[0], f1_ref[0], None,
                    w1_ref[...], b1_ref[...])
    o1_ref[0] = sf1
    d2 = _pairwise_d2(x2t_ref[0], x1c_ref[0])
    f2 = _maxgather(sf1, d2, rg2sq, ns)
    sf2 = _rnn_core(rc2sq, 2 * ns, x2t_ref[0], p2_ref[0], f2_ref[0], f2,
                    w2_ref[...], b2_ref[...])
    o2_ref[0] = sf2
    d2 = _pairwise_d2(x3t_ref[0], x2c_ref[0])
    f3 = _maxgather(sf2, d2, rg3sq, ns)
    sf3 = _rnn_core(rc3sq, 1 * ns, x3t_ref[0], p3_ref[0], f3_ref[0], f3,
                    w3_ref[...], b3_ref[...])
    o3_ref[0] = sf3


def _cells(cfg, samples, states, weights):
    x1c, x1t, x2c, x2t, x3c, x3t = samples
    (p1, f1), (p2, f2), (p3, f3) = states
    w1, b1, w2, b2, w3, b3 = weights
    bsz = x1t.shape[0]
    arrs = [x1t, x1c, x2t, x2c, x3t, x3c, p1, f1, p2, f2, p3, f3]
    specs = [pl.BlockSpec((1,) + a.shape[1:],
                          lambda i, nd=a.ndim: (i,) + (0,) * (nd - 1))
             for a in arrs]
    for w, b in ((w1, b1), (w2, b2), (w3, b3)):
        arrs += [w, b.reshape(-1, 1)]
        specs += [pl.BlockSpec(w.shape, lambda i: (0, 0)),
                  pl.BlockSpec((b.shape[0], 1), lambda i: (0, 0))]
    outs = [jax.ShapeDtypeStruct((bsz, w.shape[0], p.shape[2]), jnp.float32)
            for w, p in ((w1, p1), (w2, p2), (w3, p3))]
    out_specs = [pl.BlockSpec((1,) + o.shape[1:], lambda i: (i, 0, 0))
                 for o in outs]
    return pl.pallas_call(
        partial(_cells_kernel, cfg),
        grid=(bsz,),
        in_specs=specs,
        out_specs=out_specs,
        out_shape=outs,
    )(*arrs)


def _fpmlp_kernel(x2t_ref, x3c_ref, sf3_ref, sf2_ref, x1t_ref, x2c_ref,
                  sf1_ref, ft_ref, x1c_ref, fc_ref,
                  w1_ref, b1_ref, w2_ref, b2_ref, o_ref):
    l3f = jnp.concatenate(
        [_fp_core(x2t_ref[0], x3c_ref[0], sf3_ref[0]), sf2_ref[0]], axis=0)
    l2f = jnp.concatenate(
        [_fp_core(x1t_ref[0], x2c_ref[0], l3f), sf1_ref[0]], axis=0)
    l1f = _fp_core(ft_ref[0], x1c_ref[0], l2f)
    h = jnp.maximum(_dot(w1_ref[...], l1f, (([1], [0]))) + b1_ref[...], 0.0)
    mo = _dot(w2_ref[...], h, (([1], [0]))) + b2_ref[...]
    o_ref[0] = fc_ref[0] + mo


def _fpmlp(samples, states, frame_t, frame_c, mw1, mb1, mw2, mb2):
    x1c, x1t, x2c, x2t, _, _ = samples
    (_, sf1), (sx2c, sf2), (sx3c, sf3) = states
    bsz, _, n = frame_c.shape
    arrs = [x2t, sx3c, sf3, sf2, x1t, sx2c, sf1, frame_t, x1c, frame_c]
    specs = [pl.BlockSpec((1,) + a.shape[1:],
                          lambda i, nd=a.ndim: (i,) + (0,) * (nd - 1))
             for a in arrs]
    arrs += [mw1, mb1.reshape(-1, 1), mw2, mb2.reshape(-1, 1)]
    specs += [pl.BlockSpec(mw1.shape, lambda i: (0, 0)),
              pl.BlockSpec((mb1.shape[0], 1), lambda i: (0, 0)),
              pl.BlockSpec(mw2.shape, lambda i: (0, 0)),
              pl.BlockSpec((mb2.shape[0], 1), lambda i: (0, 0))]
    return pl.pallas_call(
        _fpmlp_kernel,
        grid=(bsz,),
        in_specs=specs,
        out_specs=pl.BlockSpec((1, 3, n), lambda i: (i, 0, 0)),
        out_shape=jax.ShapeDtypeStruct((bsz, 3, n), jnp.float32),
    )(*arrs)


def _ct(pts_c):
    return jnp.transpose(pts_c, (0, 2, 1))


def kernel(xyzs, e1W, e1b, e2W, e2b, e3W, e3b, d1W, d1b, d2W, d2b, d3W, d3b,
           mW1, mb1, mW2, mb2):
    r = RADIUS
    ns = NUM_SAMPLES
    rg2 = 2 * r / 4 + 1e-6
    rg3 = 4 * r / 4 + 1e-6
    rc1 = 1 * r + 1e-6
    rc2 = 2 * r + 1e-6
    rc3 = 3 * r + 1e-6
    cfg = (rc1 * rc1, rc2 * rc2, rc3 * rc3, rg2 * rg2, rg3 * rg3, ns)
    bsz, l, n, _ = xyzs.shape
    n1, n2, n3 = n // SUB, n // SUB // SUB, n // SUB // SUB // SUB

    def zero_states(samples):
        x1c, _, x2c, _, x3c, _ = samples
        return ((x1c, jnp.zeros((bsz, e1W.shape[0], n1), jnp.float32)),
                (x2c, jnp.zeros((bsz, e2W.shape[0], n2), jnp.float32)),
                (x3c, jnp.zeros((bsz, e3W.shape[0], n3), jnp.float32)))

    def step(samples, states, weights):
        sf1, sf2, sf3 = _cells(cfg, samples, states, weights)
        x1c, _, x2c, _, x3c, _ = samples
        return ((x1c, sf1), (x2c, sf2), (x3c, sf3))

    l2 = l // 2
    enc_c = jnp.transpose(xyzs[:, :l2], (0, 1, 3, 2)).reshape(bsz * l2, 3, n)
    x1c_a, x1t_a = _fps(enc_c, n1)
    x2c_a, x2t_a = _fps(x1c_a, n2)
    x3c_a, x3t_a = _fps(x2c_a, n3)

    def sl(a, t):
        return a.reshape((bsz, l2) + a.shape[1:])[:, t]

    enc_w = (e1W, e1b, e2W, e2b, e3W, e3b)
    dec_w = (d1W, d1b, d2W, d2b, d3W, d3b)
    states = None
    for t in range(l2):
        samples = (sl(x1c_a, t), sl(x1t_a, t), sl(x2c_a, t), sl(x2t_a, t),
                   sl(x3c_a, t), sl(x3t_a, t))
        if states is None:
            states = zero_states(samples)
        states = step(samples, states, enc_w)

    preds = []
    frame_c = jnp.transpose(xyzs[:, l2 - 1], (0, 2, 1))
    for t in range(l2, l):
        x1c, x1t = _fps(frame_c, n1)
        x2c, x2t = _fps(x1c, n2)
        x3c, x3t = _fps(x2c, n3)
        samples = (x1c, x1t, x2c, x2t, x3c, x3t)
        states = step(samples, states, dec_w)
        frame_c = _fpmlp(samples, states, _ct(frame_c), frame_c,
                         mW1, mb1, mW2, mb2)
        preds.append(_ct(frame_c))
    return jnp.stack(preds, axis=1)


# speedup vs baseline: 17.8723x; 1.1262x over previous
import jax
import jax.numpy as jnp
from functools import partial
from jax.experimental import pallas as pl

RADIUS = 4.0
NUM_SAMPLES = 4
SUB = 2

_HI = jax.lax.Precision.HIGHEST


def _dot(a, b, dims):
    return jax.lax.dot_general(a, b, (dims, ((), ())), precision=_HI,
                               preferred_element_type=jnp.float32)


def _pairwise_d2(qt, sc):
    return ((qt[:, 0:1] - sc[0:1, :]) ** 2
            + (qt[:, 1:2] - sc[1:2, :]) ** 2
            + (qt[:, 2:3] - sc[2:3, :]) ** 2)


def _select_onehots(d2, radius2, nsample):
    sq, ss = d2.shape
    iota = jax.lax.broadcasted_iota(jnp.int32, (sq, ss), 1)
    maskf = jnp.where(d2 < radius2, 1.0, 0.0)
    tri_i = jax.lax.broadcasted_iota(jnp.int32, (ss, ss), 0)
    tri_j = jax.lax.broadcasted_iota(jnp.int32, (ss, ss), 1)
    tri = jnp.where(tri_i <= tri_j, 1.0, 0.0)
    rank_incl = _dot(maskf, tri, (([1], [0])))
    rank = jnp.where(maskf > 0.0, rank_incl - maskf, -1.0)
    count = rank_incl[:, ss - 1:ss]
    e0 = jnp.where(iota == 0, 1.0, 0.0)
    oh0 = jnp.where(rank == 0.0, 1.0, 0.0) \
        + jnp.where(count > 0.0, 0.0, 1.0) * e0
    onehots = [oh0]
    for k in range(1, nsample):
        ohk = jnp.where(rank == float(k), 1.0, 0.0)
        validf = jnp.where(count > float(k), 1.0, 0.0)
        onehots.append(ohk + (1.0 - validf) * oh0)
    return onehots


def _maxgather(feats, d2, radius2, nsample):
    m = None
    for oh in _select_onehots(d2, radius2, nsample):
        gk = _dot(feats, oh, (([1], [1])))
        m = gk if m is None else jnp.maximum(m, gk)
    return m


def _rnn_core(radius2, nsample, p1t, p2c, s2, x1, w, b):
    cx = 0 if x1 is None else x1.shape[0]
    wd = w[:, 0:3]
    ws = w[:, 3 + cx:]
    g = _dot(wd, p2c, (([1], [0]))) + _dot(ws, s2, (([1], [0])))
    h = _dot(wd, p1t, (([1], [1])))
    d2 = _pairwise_d2(p1t, p2c)
    out = _maxgather(g, d2, radius2, nsample) - h + b
    if x1 is not None:
        out = out + _dot(w[:, 3:3 + cx], x1, (([1], [0])))
    return out


def _fp_core(ut, kc, kf):
    d2 = _pairwise_d2(ut, kc)
    su, sk = d2.shape
    iota = jax.lax.broadcasted_iota(jnp.int32, (su, sk), 1)
    wacc = jnp.zeros((su, sk), jnp.float32)
    rsum = jnp.zeros((su, 1), jnp.float32)
    for _ in range(3):
        mv = jnp.min(d2, axis=1, keepdims=True)
        sel = jnp.min(jnp.where(d2 == mv, iota, sk), axis=1, keepdims=True)
        oh = iota == sel
        dist = jnp.sqrt(jnp.maximum(mv, 1e-12))
        recip = 1.0 / (dist + 1e-8)
        wacc = wacc + jnp.where(oh, recip, 0.0)
        rsum = rsum + recip
        d2 = jnp.where(oh, 1e30, d2)
    wacc = wacc / rsum
    return _dot(kf, wacc, (([1], [1])))


def _fps_kernel(npoint, x_ref, o_ref):
    m_, _, n = x_ref.shape
    x0 = x_ref[:, 0, :]
    x1 = x_ref[:, 1, :]
    x2 = x_ref[:, 2, :]
    iota = jax.lax.broadcasted_iota(jnp.int32, (m_, n), 1)

    def body(i, carry):
        dists, far = carry
        onehot = iota == far
        s0 = jnp.sum(jnp.where(onehot, x0, 0.0), axis=1, keepdims=True)
        s1 = jnp.sum(jnp.where(onehot, x1, 0.0), axis=1, keepdims=True)
        s2 = jnp.sum(jnp.where(onehot, x2, 0.0), axis=1, keepdims=True)
        o_ref[pl.ds(i, 1)] = jnp.concatenate([s0, s1, s2], axis=1)[None]
        d = (x0 - s0) ** 2 + (x1 - s1) ** 2 + (x2 - s2) ** 2
        dists = jnp.minimum(dists, d)
        far = jnp.argmax(dists, axis=1, keepdims=True).astype(jnp.int32)
        return dists, far

    dists0 = jnp.full((m_, n), 1e10, jnp.float32)
    far0 = jnp.zeros((m_, 1), jnp.int32)
    if npoint % 2 == 0:
        jax.lax.fori_loop(0, npoint // 2, lambda j, c: body(2 * j + 1, body(2 * j, c)),
                          (dists0, far0))
    else:
        jax.lax.fori_loop(0, npoint, body, (dists0, far0))


def _fps(pts_c, npoint):
    m_, _, n = pts_c.shape
    out = pl.pallas_call(
        partial(_fps_kernel, npoint),
        out_shape=jax.ShapeDtypeStruct((npoint, m_, 3), jnp.float32),
    )(pts_c)
    return jnp.transpose(out, (1, 2, 0)), jnp.transpose(out, (1, 0, 2))


def _cells_kernel(cfg, x1t_ref, x1c_ref, x2t_ref, x2c_ref, x3t_ref, x3c_ref,
                  p1_ref, f1_ref, p2_ref, f2_ref, p3_ref, f3_ref,
                  w1_ref, b1_ref, w2_ref, b2_ref, w3_ref, b3_ref,
                  o1_ref, o2_ref, o3_ref):
    rc1sq, rc2sq, rc3sq, rg2sq, rg3sq, ns = cfg
    sf1 = _rnn_core(rc1sq, 3 * ns, x1t_ref[0], p1_ref[0], f1_ref[0], None,
                    w1_ref[...], b1_ref[...])
    o1_ref[0] = sf1
    d2 = _pairwise_d2(x2t_ref[0], x1c_ref[0])
    f2 = _maxgather(sf1, d2, rg2sq, ns)
    sf2 = _rnn_core(rc2sq, 2 * ns, x2t_ref[0], p2_ref[0], f2_ref[0], f2,
                    w2_ref[...], b2_ref[...])
    o2_ref[0] = sf2
    d2 = _pairwise_d2(x3t_ref[0], x2c_ref[0])
    f3 = _maxgather(sf2, d2, rg3sq, ns)
    sf3 = _rnn_core(rc3sq, 1 * ns, x3t_ref[0], p3_ref[0], f3_ref[0], f3,
                    w3_ref[...], b3_ref[...])
    o3_ref[0] = sf3


def _cells(cfg, samples, states, weights):
    x1c, x1t, x2c, x2t, x3c, x3t = samples
    (p1, f1), (p2, f2), (p3, f3) = states
    w1, b1, w2, b2, w3, b3 = weights
    bsz = x1t.shape[0]
    arrs = [x1t, x1c, x2t, x2c, x3t, x3c, p1, f1, p2, f2, p3, f3]
    specs = [pl.BlockSpec((1,) + a.shape[1:],
                          lambda i, nd=a.ndim: (i,) + (0,) * (nd - 1))
             for a in arrs]
    for w, b in ((w1, b1), (w2, b2), (w3, b3)):
        arrs += [w, b.reshape(-1, 1)]
        specs += [pl.BlockSpec(w.shape, lambda i: (0, 0)),
                  pl.BlockSpec((b.shape[0], 1), lambda i: (0, 0))]
    outs = [jax.ShapeDtypeStruct((bsz, w.shape[0], p.shape[2]), jnp.float32)
            for w, p in ((w1, p1), (w2, p2), (w3, p3))]
    out_specs = [pl.BlockSpec((1,) + o.shape[1:], lambda i: (i, 0, 0))
                 for o in outs]
    return pl.pallas_call(
        partial(_cells_kernel, cfg),
        grid=(bsz,),
        in_specs=specs,
        out_specs=out_specs,
        out_shape=outs,
    )(*arrs)


def _fpmlp_kernel(x2t_ref, x3c_ref, sf3_ref, sf2_ref, x1t_ref, x2c_ref,
                  sf1_ref, ft_ref, x1c_ref, fc_ref,
                  w1_ref, b1_ref, w2_ref, b2_ref, o_ref):
    l3f = jnp.concatenate(
        [_fp_core(x2t_ref[0], x3c_ref[0], sf3_ref[0]), sf2_ref[0]], axis=0)
    l2f = jnp.concatenate(
        [_fp_core(x1t_ref[0], x2c_ref[0], l3f), sf1_ref[0]], axis=0)
    l1f = _fp_core(ft_ref[0], x1c_ref[0], l2f)
    h = jnp.maximum(_dot(w1_ref[...], l1f, (([1], [0]))) + b1_ref[...], 0.0)
    mo = _dot(w2_ref[...], h, (([1], [0]))) + b2_ref[...]
    o_ref[0] = fc_ref[0] + mo


def _fpmlp(samples, states, frame_t, frame_c, mw1, mb1, mw2, mb2):
    x1c, x1t, x2c, x2t, _, _ = samples
    (_, sf1), (sx2c, sf2), (sx3c, sf3) = states
    bsz, _, n = frame_c.shape
    arrs = [x2t, sx3c, sf3, sf2, x1t, sx2c, sf1, frame_t, x1c, frame_c]
    specs = [pl.BlockSpec((1,) + a.shape[1:],
                          lambda i, nd=a.ndim: (i,) + (0,) * (nd - 1))
             for a in arrs]
    arrs += [mw1, mb1.reshape(-1, 1), mw2, mb2.reshape(-1, 1)]
    specs += [pl.BlockSpec(mw1.shape, lambda i: (0, 0)),
              pl.BlockSpec((mb1.shape[0], 1), lambda i: (0, 0)),
              pl.BlockSpec(mw2.shape, lambda i: (0, 0)),
              pl.BlockSpec((mb2.shape[0], 1), lambda i: (0, 0))]
    return pl.pallas_call(
        _fpmlp_kernel,
        grid=(bsz,),
        in_specs=specs,
        out_specs=pl.BlockSpec((1, 3, n), lambda i: (i, 0, 0)),
        out_shape=jax.ShapeDtypeStruct((bsz, 3, n), jnp.float32),
    )(*arrs)


def _ct(pts_c):
    return jnp.transpose(pts_c, (0, 2, 1))


def kernel(xyzs, e1W, e1b, e2W, e2b, e3W, e3b, d1W, d1b, d2W, d2b, d3W, d3b,
           mW1, mb1, mW2, mb2):
    r = RADIUS
    ns = NUM_SAMPLES
    rg2 = 2 * r / 4 + 1e-6
    rg3 = 4 * r / 4 + 1e-6
    rc1 = 1 * r + 1e-6
    rc2 = 2 * r + 1e-6
    rc3 = 3 * r + 1e-6
    cfg = (rc1 * rc1, rc2 * rc2, rc3 * rc3, rg2 * rg2, rg3 * rg3, ns)
    bsz, l, n, _ = xyzs.shape
    n1, n2, n3 = n // SUB, n // SUB // SUB, n // SUB // SUB // SUB

    def zero_states(samples):
        x1c, _, x2c, _, x3c, _ = samples
        return ((x1c, jnp.zeros((bsz, e1W.shape[0], n1), jnp.float32)),
                (x2c, jnp.zeros((bsz, e2W.shape[0], n2), jnp.float32)),
                (x3c, jnp.zeros((bsz, e3W.shape[0], n3), jnp.float32)))

    def step(samples, states, weights):
        sf1, sf2, sf3 = _cells(cfg, samples, states, weights)
        x1c, _, x2c, _, x3c, _ = samples
        return ((x1c, sf1), (x2c, sf2), (x3c, sf3))

    l2 = l // 2
    enc_c = jnp.transpose(xyzs[:, :l2], (0, 1, 3, 2)).reshape(bsz * l2, 3, n)
    x1c_a, x1t_a = _fps(enc_c, n1)
    x2c_a, x2t_a = _fps(x1c_a, n2)
    x3c_a, x3t_a = _fps(x2c_a, n3)

    def sl(a, t):
        return a.reshape((bsz, l2) + a.shape[1:])[:, t]

    enc_w = (e1W, e1b, e2W, e2b, e3W, e3b)
    dec_w = (d1W, d1b, d2W, d2b, d3W, d3b)
    states = None
    for t in range(l2):
        samples = (sl(x1c_a, t), sl(x1t_a, t), sl(x2c_a, t), sl(x2t_a, t),
                   sl(x3c_a, t), sl(x3t_a, t))
        if states is None:
            states = zero_states(samples)
        states = step(samples, states, enc_w)

    preds = []
    frame_c = jnp.transpose(xyzs[:, l2 - 1], (0, 2, 1))
    for t in range(l2, l):
        if t == l2:
            samples = (sl(x1c_a, l2 - 1), sl(x1t_a, l2 - 1), sl(x2c_a, l2 - 1),
                       sl(x2t_a, l2 - 1), sl(x3c_a, l2 - 1), sl(x3t_a, l2 - 1))
        else:
            x1c, x1t = _fps(frame_c, n1)
            x2c, x2t = _fps(x1c, n2)
            x3c, x3t = _fps(x2c, n3)
            samples = (x1c, x1t, x2c, x2t, x3c, x3t)
        states = step(samples, states, dec_w)
        frame_c = _fpmlp(samples, states, _ct(frame_c), frame_c,
                         mW1, mb1, mW2, mb2)
        preds.append(_ct(frame_c))
    return jnp.stack(preds, axis=1)


# speedup vs baseline: 17.9776x; 1.0059x over previous
import jax
import jax.numpy as jnp
from functools import partial
from jax.experimental import pallas as pl

RADIUS = 4.0
NUM_SAMPLES = 4
SUB = 2

_HI = jax.lax.Precision.HIGHEST


def _dot(a, b, dims):
    return jax.lax.dot_general(a, b, (dims, ((), ())), precision=_HI,
                               preferred_element_type=jnp.float32)


def _pairwise_d2(qt, sc):
    return ((qt[:, 0:1] - sc[0:1, :]) ** 2
            + (qt[:, 1:2] - sc[1:2, :]) ** 2
            + (qt[:, 2:3] - sc[2:3, :]) ** 2)


def _select_onehots(d2, radius2, nsample):
    sq, ss = d2.shape
    iota = jax.lax.broadcasted_iota(jnp.int32, (sq, ss), 1)
    maskf = jnp.where(d2 < radius2, 1.0, 0.0)
    tri_i = jax.lax.broadcasted_iota(jnp.int32, (ss, ss), 0)
    tri_j = jax.lax.broadcasted_iota(jnp.int32, (ss, ss), 1)
    tri = jnp.where(tri_i <= tri_j, 1.0, 0.0)
    rank_incl = _dot(maskf, tri, (([1], [0])))
    rank = jnp.where(maskf > 0.0, rank_incl - maskf, -1.0)
    count = rank_incl[:, ss - 1:ss]
    e0 = jnp.where(iota == 0, 1.0, 0.0)
    oh0 = jnp.where(rank == 0.0, 1.0, 0.0) \
        + jnp.where(count > 0.0, 0.0, 1.0) * e0
    onehots = [oh0]
    for k in range(1, nsample):
        ohk = jnp.where(rank == float(k), 1.0, 0.0)
        validf = jnp.where(count > float(k), 1.0, 0.0)
        onehots.append(ohk + (1.0 - validf) * oh0)
    return onehots


def _maxgather(feats, d2, radius2, nsample):
    m = None
    for oh in _select_onehots(d2, radius2, nsample):
        gk = _dot(feats, oh, (([1], [1])))
        m = gk if m is None else jnp.maximum(m, gk)
    return m


def _rnn_core(radius2, nsample, p1t, p2c, s2, x1, w, b):
    cx = 0 if x1 is None else x1.shape[0]
    wd = w[:, 0:3]
    ws = w[:, 3 + cx:]
    g = _dot(wd, p2c, (([1], [0]))) + _dot(ws, s2, (([1], [0])))
    h = _dot(wd, p1t, (([1], [1])))
    d2 = _pairwise_d2(p1t, p2c)
    out = _maxgather(g, d2, radius2, nsample) - h + b
    if x1 is not None:
        out = out + _dot(w[:, 3:3 + cx], x1, (([1], [0])))
    return out


def _fp_core(ut, kc, kf):
    d2 = _pairwise_d2(ut, kc)
    su, sk = d2.shape
    iota = jax.lax.broadcasted_iota(jnp.int32, (su, sk), 1)
    wacc = jnp.zeros((su, sk), jnp.float32)
    rsum = jnp.zeros((su, 1), jnp.float32)
    for _ in range(3):
        mv = jnp.min(d2, axis=1, keepdims=True)
        sel = jnp.min(jnp.where(d2 == mv, iota, sk), axis=1, keepdims=True)
        oh = iota == sel
        dist = jnp.sqrt(jnp.maximum(mv, 1e-12))
        recip = 1.0 / (dist + 1e-8)
        wacc = wacc + jnp.where(oh, recip, 0.0)
        rsum = rsum + recip
        d2 = jnp.where(oh, 1e30, d2)
    wacc = wacc / rsum
    return _dot(kf, wacc, (([1], [1])))


def _fps_kernel(npoint, x_ref, o_ref):
    m_, _, n = x_ref.shape
    x0 = x_ref[:, 0, :]
    x1 = x_ref[:, 1, :]
    x2 = x_ref[:, 2, :]
    iota = jax.lax.broadcasted_iota(jnp.int32, (m_, n), 1)

    def body(i, carry):
        dists, far = carry
        onehot = iota == far
        s0 = jnp.sum(jnp.where(onehot, x0, 0.0), axis=1, keepdims=True)
        s1 = jnp.sum(jnp.where(onehot, x1, 0.0), axis=1, keepdims=True)
        s2 = jnp.sum(jnp.where(onehot, x2, 0.0), axis=1, keepdims=True)
        o_ref[pl.ds(i, 1)] = jnp.concatenate([s0, s1, s2], axis=1)[None]
        d = (x0 - s0) ** 2 + (x1 - s1) ** 2 + (x2 - s2) ** 2
        dists = jnp.minimum(dists, d)
        far = jnp.argmax(dists, axis=1, keepdims=True).astype(jnp.int32)
        return dists, far

    dists0 = jnp.full((m_, n), 1e10, jnp.float32)
    far0 = jnp.zeros((m_, 1), jnp.int32)
    if npoint % 4 == 0:
        def body4(j, c):
            for u in range(4):
                c = body(4 * j + u, c)
            return c
        jax.lax.fori_loop(0, npoint // 4, body4, (dists0, far0))
    elif npoint % 2 == 0:
        jax.lax.fori_loop(0, npoint // 2, lambda j, c: body(2 * j + 1, body(2 * j, c)),
                          (dists0, far0))
    else:
        jax.lax.fori_loop(0, npoint, body, (dists0, far0))


def _fps(pts_c, npoint):
    m_, _, n = pts_c.shape
    out = pl.pallas_call(
        partial(_fps_kernel, npoint),
        out_shape=jax.ShapeDtypeStruct((npoint, m_, 3), jnp.float32),
    )(pts_c)
    return jnp.transpose(out, (1, 2, 0)), jnp.transpose(out, (1, 0, 2))


def _cells_kernel(cfg, x1t_ref, x1c_ref, x2t_ref, x2c_ref, x3t_ref, x3c_ref,
                  p1_ref, f1_ref, p2_ref, f2_ref, p3_ref, f3_ref,
                  w1_ref, b1_ref, w2_ref, b2_ref, w3_ref, b3_ref,
                  o1_ref, o2_ref, o3_ref):
    rc1sq, rc2sq, rc3sq, rg2sq, rg3sq, ns = cfg
    sf1 = _rnn_core(rc1sq, 3 * ns, x1t_ref[0], p1_ref[0], f1_ref[0], None,
                    w1_ref[...], b1_ref[...])
    o1_ref[0] = sf1
    d2 = _pairwise_d2(x2t_ref[0], x1c_ref[0])
    f2 = _maxgather(sf1, d2, rg2sq, ns)
    sf2 = _rnn_core(rc2sq, 2 * ns, x2t_ref[0], p2_ref[0], f2_ref[0], f2,
                    w2_ref[...], b2_ref[...])
    o2_ref[0] = sf2
    d2 = _pairwise_d2(x3t_ref[0], x2c_ref[0])
    f3 = _maxgather(sf2, d2, rg3sq, ns)
    sf3 = _rnn_core(rc3sq, 1 * ns, x3t_ref[0], p3_ref[0], f3_ref[0], f3,
                    w3_ref[...], b3_ref[...])
    o3_ref[0] = sf3


def _cells(cfg, samples, states, weights):
    x1c, x1t, x2c, x2t, x3c, x3t = samples
    (p1, f1), (p2, f2), (p3, f3) = states
    w1, b1, w2, b2, w3, b3 = weights
    bsz = x1t.shape[0]
    arrs = [x1t, x1c, x2t, x2c, x3t, x3c, p1, f1, p2, f2, p3, f3]
    specs = [pl.BlockSpec((1,) + a.shape[1:],
                          lambda i, nd=a.ndim: (i,) + (0,) * (nd - 1))
             for a in arrs]
    for w, b in ((w1, b1), (w2, b2), (w3, b3)):
        arrs += [w, b.reshape(-1, 1)]
        specs += [pl.BlockSpec(w.shape, lambda i: (0, 0)),
                  pl.BlockSpec((b.shape[0], 1), lambda i: (0, 0))]
    outs = [jax.ShapeDtypeStruct((bsz, w.shape[0], p.shape[2]), jnp.float32)
            for w, p in ((w1, p1), (w2, p2), (w3, p3))]
    out_specs = [pl.BlockSpec((1,) + o.shape[1:], lambda i: (i, 0, 0))
                 for o in outs]
    return pl.pallas_call(
        partial(_cells_kernel, cfg),
        grid=(bsz,),
        in_specs=specs,
        out_specs=out_specs,
        out_shape=outs,
    )(*arrs)


def _fpmlp_kernel(x2t_ref, x3c_ref, sf3_ref, sf2_ref, x1t_ref, x2c_ref,
                  sf1_ref, ft_ref, x1c_ref, fc_ref,
                  w1_ref, b1_ref, w2_ref, b2_ref, o_ref):
    l3f = jnp.concatenate(
        [_fp_core(x2t_ref[0], x3c_ref[0], sf3_ref[0]), sf2_ref[0]], axis=0)
    l2f = jnp.concatenate(
        [_fp_core(x1t_ref[0], x2c_ref[0], l3f), sf1_ref[0]], axis=0)
    l1f = _fp_core(ft_ref[0], x1c_ref[0], l2f)
    h = jnp.maximum(_dot(w1_ref[...], l1f, (([1], [0]))) + b1_ref[...], 0.0)
    mo = _dot(w2_ref[...], h, (([1], [0]))) + b2_ref[...]
    o_ref[0] = fc_ref[0] + mo


def _fpmlp(samples, states, frame_t, frame_c, mw1, mb1, mw2, mb2):
    x1c, x1t, x2c, x2t, _, _ = samples
    (_, sf1), (sx2c, sf2), (sx3c, sf3) = states
    bsz, _, n = frame_c.shape
    arrs = [x2t, sx3c, sf3, sf2, x1t, sx2c, sf1, frame_t, x1c, frame_c]
    specs = [pl.BlockSpec((1,) + a.shape[1:],
                          lambda i, nd=a.ndim: (i,) + (0,) * (nd - 1))
             for a in arrs]
    arrs += [mw1, mb1.reshape(-1, 1), mw2, mb2.reshape(-1, 1)]
    specs += [pl.BlockSpec(mw1.shape, lambda i: (0, 0)),
              pl.BlockSpec((mb1.shape[0], 1), lambda i: (0, 0)),
              pl.BlockSpec(mw2.shape, lambda i: (0, 0)),
              pl.BlockSpec((mb2.shape[0], 1), lambda i: (0, 0))]
    return pl.pallas_call(
        _fpmlp_kernel,
        grid=(bsz,),
        in_specs=specs,
        out_specs=pl.BlockSpec((1, 3, n), lambda i: (i, 0, 0)),
        out_shape=jax.ShapeDtypeStruct((bsz, 3, n), jnp.float32),
    )(*arrs)


def _ct(pts_c):
    return jnp.transpose(pts_c, (0, 2, 1))


def kernel(xyzs, e1W, e1b, e2W, e2b, e3W, e3b, d1W, d1b, d2W, d2b, d3W, d3b,
           mW1, mb1, mW2, mb2):
    r = RADIUS
    ns = NUM_SAMPLES
    rg2 = 2 * r / 4 + 1e-6
    rg3 = 4 * r / 4 + 1e-6
    rc1 = 1 * r + 1e-6
    rc2 = 2 * r + 1e-6
    rc3 = 3 * r + 1e-6
    cfg = (rc1 * rc1, rc2 * rc2, rc3 * rc3, rg2 * rg2, rg3 * rg3, ns)
    bsz, l, n, _ = xyzs.shape
    n1, n2, n3 = n // SUB, n // SUB // SUB, n // SUB // SUB // SUB

    def zero_states(samples):
        x1c, _, x2c, _, x3c, _ = samples
        return ((x1c, jnp.zeros((bsz, e1W.shape[0], n1), jnp.float32)),
                (x2c, jnp.zeros((bsz, e2W.shape[0], n2), jnp.float32)),
                (x3c, jnp.zeros((bsz, e3W.shape[0], n3), jnp.float32)))

    def step(samples, states, weights):
        sf1, sf2, sf3 = _cells(cfg, samples, states, weights)
        x1c, _, x2c, _, x3c, _ = samples
        return ((x1c, sf1), (x2c, sf2), (x3c, sf3))

    l2 = l // 2
    enc_c = jnp.transpose(xyzs[:, :l2], (0, 1, 3, 2)).reshape(bsz * l2, 3, n)
    x1c_a, x1t_a = _fps(enc_c, n1)
    x2c_a, x2t_a = _fps(x1c_a, n2)
    x3c_a, x3t_a = _fps(x2c_a, n3)

    def sl(a, t):
        return a.reshape((bsz, l2) + a.shape[1:])[:, t]

    enc_w = (e1W, e1b, e2W, e2b, e3W, e3b)
    dec_w = (d1W, d1b, d2W, d2b, d3W, d3b)
    states = None
    for t in range(l2):
        samples = (sl(x1c_a, t), sl(x1t_a, t), sl(x2c_a, t), sl(x2t_a, t),
                   sl(x3c_a, t), sl(x3t_a, t))
        if states is None:
            states = zero_states(samples)
        states = step(samples, states, enc_w)

    preds = []
    frame_c = jnp.transpose(xyzs[:, l2 - 1], (0, 2, 1))
    for t in range(l2, l):
        if t == l2:
            samples = (sl(x1c_a, l2 - 1), sl(x1t_a, l2 - 1), sl(x2c_a, l2 - 1),
                       sl(x2t_a, l2 - 1), sl(x3c_a, l2 - 1), sl(x3t_a, l2 - 1))
        else:
            x1c, x1t = _fps(frame_c, n1)
            x2c, x2t = _fps(x1c, n2)
            x3c, x3t = _fps(x2c, n3)
            samples = (x1c, x1t, x2c, x2t, x3c, x3t)
        states = step(samples, states, dec_w)
        frame_c = _fpmlp(samples, states, _ct(frame_c), frame_c,
                         mW1, mb1, mW2, mb2)
        preds.append(_ct(frame_c))
    return jnp.stack(preds, axis=1)
